# Initial kernel scaffold; baseline (speedup 1.0000x reference)
import jax
import jax.numpy as jnp
from jax.experimental import pallas as pl


def kernel(state, species_table, item_table, ability_table, move_table, W, b):
    raise NotImplementedError("write your pallas kernel here")


# speedup vs baseline: 1.1607x; 1.1607x over previous
import functools

import jax
import jax.numpy as jnp
from jax import lax
from jax.experimental import pallas as pl
from jax.experimental.pallas import tpu as pltpu
from jax.experimental.pallas import tpu_sc as plsc

B = 16384
EMB = 32
NCOLS = 11
OTHERS = 128
NW = 32
BPW = B // NW
CHUNK = 128
NCHUNK = BPW // CHUNK


def _sc_gather(idx, species, item, ability, move):
    mesh = plsc.VectorSubcoreMesh(core_axis_name="c", subcore_axis_name="s")

    @functools.partial(
        pl.kernel,
        out_type=jax.ShapeDtypeStruct((NCOLS, B, EMB), jnp.float32),
        mesh=mesh,
        scratch_types=[
            pltpu.VMEM((NCOLS * BPW,), jnp.int32),
            pltpu.VMEM((BPW, EMB), jnp.float32),
            pltpu.VMEM((BPW, EMB), jnp.float32),
            pltpu.SemaphoreType.DMA,
            pltpu.SemaphoreType.DMA,
            pltpu.SemaphoreType.DMA,
        ],
        compiler_params=pltpu.CompilerParams(use_tc_tiling_on_sc=False),
    )
    def k(sp_hbm, it_hbm, ab_hbm, mv_hbm, idx_hbm, out_hbm,
          idx_v, buf0, buf1, gsem, wsem0, wsem1):
        wid = lax.axis_index("s") * 2 + lax.axis_index("c")
        base = wid * BPW
        idx_cps = [pltpu.async_copy(
            idx_hbm.at[pl.ds(c * B + base, BPW)],
            idx_v.at[pl.ds(c * BPW, BPW)], gsem) for c in range(NCOLS)]
        for cp in idx_cps:
            cp.wait()
        tables = [sp_hbm, it_hbm, ab_hbm, ab_hbm, ab_hbm, ab_hbm, ab_hbm,
                  mv_hbm, mv_hbm, mv_hbm, mv_hbm]
        bufs = [buf0, buf1]
        wsems = [wsem0, wsem1]
        pending = [None, None]
        for c in range(NCOLS):
            p = c % 2
            buf = bufs[p]
            if pending[p] is not None:
                pending[p].wait()
            gathers = []
            for j in range(NCHUNK):
                gathers.append(pltpu.async_copy(
                    tables[c].at[idx_v.at[pl.ds(c * BPW + j * CHUNK, CHUNK)]],
                    buf.at[pl.ds(j * CHUNK, CHUNK)],
                    gsem))
            for cp in gathers:
                cp.wait()
            pending[p] = pltpu.async_copy(
                buf, out_hbm.at[c, pl.ds(base, BPW)], wsems[p])
        for p in range(2):
            if pending[p] is not None:
                pending[p].wait()

    return k(species, item, ability, move, idx)


def _tc_head(emb, others, w2, b2):
    bm = 2048

    def body(emb_ref, oth_ref, w2_ref, b_ref, out_ref):
        acc = jnp.dot(oth_ref[...], w2_ref[NCOLS * EMB:, :],
                      preferred_element_type=jnp.float32)
        for c in range(NCOLS):
            acc = acc + jnp.dot(emb_ref[c], w2_ref[c * EMB:(c + 1) * EMB, :],
                                preferred_element_type=jnp.float32)
        out_ref[...] = jnp.maximum(acc + b_ref[0], 0.0)

    return pl.pallas_call(
        body,
        grid=(B // bm,),
        in_specs=[
            pl.BlockSpec((NCOLS, bm, EMB), lambda i: (0, i, 0)),
            pl.BlockSpec((bm, OTHERS), lambda i: (i, 0)),
            pl.BlockSpec((NCOLS * EMB + OTHERS, EMB), lambda i: (0, 0)),
            pl.BlockSpec((1, EMB), lambda i: (0, 0)),
        ],
        out_specs=pl.BlockSpec((bm, EMB), lambda i: (i, 0)),
        out_shape=jax.ShapeDtypeStruct((B, EMB), jnp.float32),
    )(emb, others, w2, b2)


def kernel(state, species_table, item_table, ability_table, move_table, W, b):
    idx = state[:, :NCOLS].astype(jnp.int32).T.reshape(-1)
    others = state[:, NCOLS:]
    Wt = W.T
    w2 = jnp.concatenate([
        Wt[0:3 * EMB],
        jnp.tile(Wt[3 * EMB:4 * EMB] * 0.25, (4, 1)),
        jnp.tile(Wt[4 * EMB:5 * EMB] * 0.25, (4, 1)),
        Wt[5 * EMB:],
    ], axis=0)
    emb = _sc_gather(idx, species_table, item_table, ability_table, move_table)
    return _tc_head(emb, others, w2, b.reshape(1, EMB))
